# baseline (speedup 1.0000x reference)
import functools

import jax
import jax.numpy as jnp
from jax import lax
from jax.experimental import pallas as pl
from jax.experimental.pallas import tpu as pltpu

_F = 16
_K = 20
_H = 1


def _pair_body(N, K, G, FD, L1, OUT, H, *refs):
    (feats_ref, gt_ref, wpre_ref, bpre_ref, wcat_ref, w2_ref, b2_ref,
     a2_ref, ab2_ref, lng_ref, lnb_ref, out_ref) = refs[:12]
    drefs = refs[12:12 + H]
    f32 = jnp.float32
    rows = lax.broadcasted_iota(jnp.int32, (N, N), 0)
    cols = lax.broadcasted_iota(jnp.int32, (N, N), 1)

    featsH = [feats_ref[h * N:(h + 1) * N, :] for h in range(H)]

    def build_d2(h):
        feats = featsH[h]
        d2 = jnp.zeros((N, N), f32)
        for c in range(G):
            col = feats[:, c:c + 1]
            row = gt_ref[c:c + 1, h * N:(h + 1) * N]
            dif = col - row
            d2 = d2 + dif * dif
        return jnp.where(rows == cols, d2 + 1e10, d2)

    preH = []
    for h in range(H):
        drefs[h][...] = build_d2(h)
        preH.append(jnp.dot(featsH[h], wpre_ref[...],
                            preferred_element_type=f32) + bpre_ref[...])

    def mlp(pre, nbr, acc):
        t = jax.nn.relu(pre + jnp.dot(nbr, wcat_ref[...],
                                      preferred_element_type=f32))
        h = t[:, :L1]
        ha = t[:, L1:]
        ek = jax.nn.relu(jnp.dot(h, w2_ref[...], preferred_element_type=f32)
                         + b2_ref[...])
        gb = jnp.dot(ha, a2_ref[...], preferred_element_type=f32) + ab2_ref[...]
        gam = jax.nn.sigmoid(gb[:, :OUT] + 1.0)
        bet = gb[:, OUT:]
        return jnp.maximum(acc, gam * ek + bet)

    def select(h, cnt, lastv):
        dv = drefs[h][...]
        cand = jnp.where(dv > lastv, dv, 3e38)
        rmin = jnp.min(cand, axis=1, keepdims=True)
        m = cand == rmin
        nbr = jnp.dot(m.astype(f32), featsH[h], preferred_element_type=f32)
        return nbr, cnt + nbr[:, FD:FD + 1], rmin

    acc0 = jnp.full((N, OUT), -jnp.inf, f32)
    state = []
    for h in range(H):
        nbr, cnt, lastv = select(h, jnp.zeros((N, 1), f32),
                                 jnp.full((N, 1), -1.0, f32))
        state += [acc0, cnt, nbr, lastv]

    def fast_step(_, st):
        out = []
        nxt = []
        for h in range(H):
            acc, cnt, nbr_prev, lastv = st[4 * h:4 * h + 4]
            nbr, cnt, lastv = select(h, cnt, lastv)
            nxt.append((mlp(preH[h], nbr_prev, acc), cnt, nbr, lastv))
        for quad in nxt:
            out += list(quad)
        return tuple(out)

    st = lax.fori_loop(1, K, fast_step, tuple(state), unroll=2)

    for h in range(H):
        acc, cnt, nbr_prev, _ = st[4 * h:4 * h + 4]
        acc = mlp(preH[h], nbr_prev, acc)
        total = jnp.sum(cnt)

        def exact(h=h):
            drefs[h][...] = build_d2(h)

            def step(_, a):
                dv = drefs[h][...]
                rmin = jnp.min(dv, axis=1, keepdims=True)
                idx = jnp.where(dv == rmin, cols, N)
                amin = jnp.min(idx, axis=1, keepdims=True)
                onehot = cols == amin
                drefs[h][...] = jnp.where(onehot, 3e38, dv)
                nbr = jnp.dot(onehot.astype(f32), featsH[h],
                              preferred_element_type=f32)
                return mlp(preH[h], nbr, a)

            return lax.fori_loop(0, K, step, acc0)

        acc = lax.cond(total == float(N * K), lambda acc=acc: acc, exact)

        mu = jnp.mean(acc, axis=1, keepdims=True)
        xc = acc - mu
        var = jnp.mean(xc * xc, axis=1, keepdims=True)
        y = xc * lax.rsqrt(var + 1e-5) * lng_ref[...] + lnb_ref[...]
        out_ref[h * N:(h + 1) * N, :] = jax.nn.relu(y)


def _edgeconv(geom, aux, W1, b1, W2, b2, A1, ab1, A2, ab2, ln_g, ln_b,
              frames, k, interleave):
    P, G = geom.shape
    A = aux.shape[1]
    N = P // frames
    FD = G + A
    L1 = W1.shape[1]
    HA = A1.shape[1]
    OUT = W2.shape[1]
    TW = L1 + HA
    H = interleave

    f32 = jnp.float32
    feats = jnp.concatenate(
        [geom, aux, jnp.ones((P, 1), f32)], axis=1)
    geomT = geom.T
    wpre = jnp.zeros((FD + 1, TW), f32)
    wpre = wpre.at[:G, :L1].set(W1[:G] - W1[G:])
    wpre = wpre.at[G:FD, L1:].set(A1[:A])
    wcat = jnp.zeros((FD + 1, TW), f32)
    wcat = wcat.at[:G, :L1].set(W1[G:])
    wcat = wcat.at[G:FD, L1:].set(A1[A:])
    bpre = jnp.concatenate([b1, ab1]).reshape(1, TW)

    body = functools.partial(_pair_body, N, k, G, FD, L1, OUT, H)
    full = lambda i: (0, 0)
    out = pl.pallas_call(
        body,
        grid=(frames // H,),
        in_specs=[
            pl.BlockSpec((H * N, FD + 1), lambda i: (i, 0)),
            pl.BlockSpec((G, H * N), lambda i: (0, i)),
            pl.BlockSpec((FD + 1, TW), full),
            pl.BlockSpec((1, TW), full),
            pl.BlockSpec((FD + 1, TW), full),
            pl.BlockSpec((L1, OUT), full),
            pl.BlockSpec((1, OUT), full),
            pl.BlockSpec((HA, 2 * OUT), full),
            pl.BlockSpec((1, 2 * OUT), full),
            pl.BlockSpec((1, OUT), full),
            pl.BlockSpec((1, OUT), full),
        ],
        out_specs=pl.BlockSpec((H * N, OUT), lambda i: (i, 0)),
        out_shape=jax.ShapeDtypeStruct((P, OUT), f32),
        scratch_shapes=[pltpu.VMEM((N, N), f32) for _ in range(H)],
    )(feats, geomT, wpre, bpre, wcat, W2, b2.reshape(1, OUT), A2,
      ab2.reshape(1, 2 * OUT), ln_g.reshape(1, OUT), ln_b.reshape(1, OUT))
    return out


def kernel(geom, aux, batch, W1, b1, W2, b2, A1, ab1, A2, ab2, ln_g, ln_b):
    del batch
    return _edgeconv(geom, aux, W1, b1, W2, b2, A1, ab1, A2, ab2,
                     ln_g, ln_b, _F, _K, _H)


# speedup vs baseline: 1.0204x; 1.0204x over previous
import functools

import jax
import jax.numpy as jnp
from jax import lax
from jax.experimental import pallas as pl
from jax.experimental.pallas import tpu as pltpu

_F = 16
_K = 20
_H = 1


def _pair_body(N, K, G, FD, L1, OUT, H, *refs):
    (feats_ref, gt_ref, wpre_ref, bpre_ref, wcat_ref, w2_ref, b2_ref,
     a2_ref, ab2_ref, lng_ref, lnb_ref, out_ref) = refs[:12]
    drefs = refs[12:12 + H]
    f32 = jnp.float32
    rows = lax.broadcasted_iota(jnp.int32, (N, N), 0)
    cols = lax.broadcasted_iota(jnp.int32, (N, N), 1)

    featsH = [feats_ref[h * N:(h + 1) * N, :] for h in range(H)]

    def build_d2(h):
        feats = featsH[h]
        d2 = jnp.zeros((N, N), f32)
        for c in range(G):
            col = feats[:, c:c + 1]
            row = gt_ref[c:c + 1, h * N:(h + 1) * N]
            dif = col - row
            d2 = d2 + dif * dif
        return jnp.where(rows == cols, d2 + 1e10, d2)

    preH = []
    rmin0H = []
    for h in range(H):
        d2v = build_d2(h)
        drefs[h][...] = d2v
        rmin0H.append(jnp.min(d2v, axis=1, keepdims=True))
        preH.append(jnp.dot(featsH[h], wpre_ref[...],
                            preferred_element_type=f32) + bpre_ref[...])

    def mlp(pre, nbr, acc):
        t = jax.nn.relu(pre + jnp.dot(nbr, wcat_ref[...],
                                      preferred_element_type=f32))
        h = t[:, :L1]
        ha = t[:, L1:]
        ek = jax.nn.relu(jnp.dot(h, w2_ref[...], preferred_element_type=f32)
                         + b2_ref[...])
        gb = jnp.dot(ha, a2_ref[...], preferred_element_type=f32) + ab2_ref[...]
        gam = jax.nn.sigmoid(gb[:, :OUT] + 1.0)
        bet = gb[:, OUT:]
        return jnp.maximum(acc, gam * ek + bet)

    def select(h, cnt, rmin):
        dv = drefs[h][...]
        m = dv == rmin
        dv_new = jnp.where(m, 3e38, dv)
        drefs[h][...] = dv_new
        rmin_next = jnp.min(dv_new, axis=1, keepdims=True)
        nbr = jnp.dot(m.astype(f32), featsH[h], preferred_element_type=f32)
        return nbr, cnt + nbr[:, FD:FD + 1], rmin_next

    acc0 = jnp.full((N, OUT), -jnp.inf, f32)
    state = []
    for h in range(H):
        nbr, cnt, rmin = select(h, jnp.zeros((N, 1), f32), rmin0H[h])
        state += [acc0, cnt, nbr, rmin]

    def fast_step(_, st):
        out = []
        nxt = []
        for h in range(H):
            acc, cnt, nbr_prev, rmin = st[4 * h:4 * h + 4]
            nbr, cnt, rmin = select(h, cnt, rmin)
            nxt.append((mlp(preH[h], nbr_prev, acc), cnt, nbr, rmin))
        for quad in nxt:
            out += list(quad)
        return tuple(out)

    st = lax.fori_loop(1, K, fast_step, tuple(state))

    for h in range(H):
        acc, cnt, nbr_prev, _ = st[4 * h:4 * h + 4]
        acc = mlp(preH[h], nbr_prev, acc)
        total = jnp.sum(cnt)

        def exact(h=h):
            drefs[h][...] = build_d2(h)

            def step(_, a):
                dv = drefs[h][...]
                rmin = jnp.min(dv, axis=1, keepdims=True)
                idx = jnp.where(dv == rmin, cols, N)
                amin = jnp.min(idx, axis=1, keepdims=True)
                onehot = cols == amin
                drefs[h][...] = jnp.where(onehot, 3e38, dv)
                nbr = jnp.dot(onehot.astype(f32), featsH[h],
                              preferred_element_type=f32)
                return mlp(preH[h], nbr, a)

            return lax.fori_loop(0, K, step, acc0)

        acc = lax.cond(total == float(N * K), lambda acc=acc: acc, exact)

        mu = jnp.mean(acc, axis=1, keepdims=True)
        xc = acc - mu
        var = jnp.mean(xc * xc, axis=1, keepdims=True)
        y = xc * lax.rsqrt(var + 1e-5) * lng_ref[...] + lnb_ref[...]
        out_ref[h * N:(h + 1) * N, :] = jax.nn.relu(y)


def _edgeconv(geom, aux, W1, b1, W2, b2, A1, ab1, A2, ab2, ln_g, ln_b,
              frames, k, interleave):
    P, G = geom.shape
    A = aux.shape[1]
    N = P // frames
    FD = G + A
    L1 = W1.shape[1]
    HA = A1.shape[1]
    OUT = W2.shape[1]
    TW = L1 + HA
    H = interleave

    f32 = jnp.float32
    feats = jnp.concatenate(
        [geom, aux, jnp.ones((P, 1), f32)], axis=1)
    geomT = geom.T
    wpre = jnp.zeros((FD + 1, TW), f32)
    wpre = wpre.at[:G, :L1].set(W1[:G] - W1[G:])
    wpre = wpre.at[G:FD, L1:].set(A1[:A])
    wcat = jnp.zeros((FD + 1, TW), f32)
    wcat = wcat.at[:G, :L1].set(W1[G:])
    wcat = wcat.at[G:FD, L1:].set(A1[A:])
    bpre = jnp.concatenate([b1, ab1]).reshape(1, TW)

    body = functools.partial(_pair_body, N, k, G, FD, L1, OUT, H)
    full = lambda i: (0, 0)
    out = pl.pallas_call(
        body,
        grid=(frames // H,),
        in_specs=[
            pl.BlockSpec((H * N, FD + 1), lambda i: (i, 0)),
            pl.BlockSpec((G, H * N), lambda i: (0, i)),
            pl.BlockSpec((FD + 1, TW), full),
            pl.BlockSpec((1, TW), full),
            pl.BlockSpec((FD + 1, TW), full),
            pl.BlockSpec((L1, OUT), full),
            pl.BlockSpec((1, OUT), full),
            pl.BlockSpec((HA, 2 * OUT), full),
            pl.BlockSpec((1, 2 * OUT), full),
            pl.BlockSpec((1, OUT), full),
            pl.BlockSpec((1, OUT), full),
        ],
        out_specs=pl.BlockSpec((H * N, OUT), lambda i: (i, 0)),
        out_shape=jax.ShapeDtypeStruct((P, OUT), f32),
        scratch_shapes=[pltpu.VMEM((N, N), f32) for _ in range(H)],
    )(feats, geomT, wpre, bpre, wcat, W2, b2.reshape(1, OUT), A2,
      ab2.reshape(1, 2 * OUT), ln_g.reshape(1, OUT), ln_b.reshape(1, OUT))
    return out


def kernel(geom, aux, batch, W1, b1, W2, b2, A1, ab1, A2, ab2, ln_g, ln_b):
    del batch
    return _edgeconv(geom, aux, W1, b1, W2, b2, A1, ab1, A2, ab2,
                     ln_g, ln_b, _F, _K, _H)


# speedup vs baseline: 1.0568x; 1.0357x over previous
import functools

import jax
import jax.numpy as jnp
from jax import lax
from jax.experimental import pallas as pl
from jax.experimental.pallas import tpu as pltpu

_F = 16
_K = 20
_H = 1


def _pair_body(N, K, G, FD, L1, OUT, H, *refs):
    (feats_ref, gt_ref, wpre_ref, bpre_ref, wcat_ref, w2_ref, b2_ref,
     a2_ref, ab2_ref, lng_ref, lnb_ref, out_ref) = refs[:12]
    drefs = refs[12:12 + H]
    f32 = jnp.float32
    rows = lax.broadcasted_iota(jnp.int32, (N, N), 0)
    cols = lax.broadcasted_iota(jnp.int32, (N, N), 1)

    featsH = [feats_ref[h * N:(h + 1) * N, :] for h in range(H)]

    def build_d2(h):
        feats = featsH[h]
        d2 = jnp.zeros((N, N), f32)
        for c in range(G):
            col = feats[:, c:c + 1]
            row = gt_ref[c:c + 1, h * N:(h + 1) * N]
            dif = col - row
            d2 = d2 + dif * dif
        return jnp.where(rows == cols, d2 + 1e10, d2)

    preH = []
    rmin0H = []
    for h in range(H):
        d2v = build_d2(h)
        drefs[h][...] = d2v
        rmin0H.append(jnp.min(d2v, axis=1, keepdims=True))
        preH.append(jnp.dot(featsH[h], wpre_ref[...],
                            preferred_element_type=f32) + bpre_ref[...])

    def mlp(pre, nbr, acc):
        t = jax.nn.relu(pre + jnp.dot(nbr, wcat_ref[...],
                                      preferred_element_type=f32))
        h = t[:, :L1]
        ha = t[:, L1:]
        ek = jax.nn.relu(jnp.dot(h, w2_ref[...], preferred_element_type=f32)
                         + b2_ref[...])
        gb = jnp.dot(ha, a2_ref[...], preferred_element_type=f32) + ab2_ref[...]
        gam = jax.nn.sigmoid(gb[:, :OUT] + 1.0)
        bet = gb[:, OUT:]
        return jnp.maximum(acc, gam * ek + bet)

    def select2(h, cnt, rmin):
        dv = drefs[h][...]
        m1 = dv == rmin
        dv2 = jnp.where(m1, 3e38, dv)
        rmin2 = jnp.min(dv2, axis=1, keepdims=True)
        m2 = dv2 == rmin2
        upd = jnp.where(m2, 3e38, dv2)
        drefs[h][...] = upd
        rmin_next = jnp.min(upd, axis=1, keepdims=True)
        nbr1 = jnp.dot(m1.astype(f32), featsH[h], preferred_element_type=f32)
        nbr2 = jnp.dot(m2.astype(f32), featsH[h], preferred_element_type=f32)
        cnt = cnt + (nbr1[:, FD:FD + 1] + nbr2[:, FD:FD + 1])
        return nbr1, nbr2, cnt, rmin_next

    acc0 = jnp.full((N, OUT), -jnp.inf, f32)
    state = []
    for h in range(H):
        nbr1, nbr2, cnt, rmin = select2(h, jnp.zeros((N, 1), f32), rmin0H[h])
        state += [acc0, cnt, nbr1, nbr2, rmin]

    def fast_step(_, st):
        out = []
        nxt = []
        for h in range(H):
            acc, cnt, nbrA, nbrB, rmin = st[5 * h:5 * h + 5]
            nbr1, nbr2, cnt, rmin = select2(h, cnt, rmin)
            acc = mlp(preH[h], nbrB, mlp(preH[h], nbrA, acc))
            nxt.append((acc, cnt, nbr1, nbr2, rmin))
        for quint in nxt:
            out += list(quint)
        return tuple(out)

    st = lax.fori_loop(1, K // 2, fast_step, tuple(state))

    for h in range(H):
        acc, cnt, nbrA, nbrB, _ = st[5 * h:5 * h + 5]
        acc = mlp(preH[h], nbrB, mlp(preH[h], nbrA, acc))
        total = jnp.sum(cnt)

        def exact(h=h):
            drefs[h][...] = build_d2(h)

            def step(_, a):
                dv = drefs[h][...]
                rmin = jnp.min(dv, axis=1, keepdims=True)
                idx = jnp.where(dv == rmin, cols, N)
                amin = jnp.min(idx, axis=1, keepdims=True)
                onehot = cols == amin
                drefs[h][...] = jnp.where(onehot, 3e38, dv)
                nbr = jnp.dot(onehot.astype(f32), featsH[h],
                              preferred_element_type=f32)
                return mlp(preH[h], nbr, a)

            return lax.fori_loop(0, K, step, acc0)

        acc = lax.cond(total == float(N * K), lambda acc=acc: acc, exact)

        mu = jnp.mean(acc, axis=1, keepdims=True)
        xc = acc - mu
        var = jnp.mean(xc * xc, axis=1, keepdims=True)
        y = xc * lax.rsqrt(var + 1e-5) * lng_ref[...] + lnb_ref[...]
        out_ref[h * N:(h + 1) * N, :] = jax.nn.relu(y)


def _edgeconv(geom, aux, W1, b1, W2, b2, A1, ab1, A2, ab2, ln_g, ln_b,
              frames, k, interleave):
    P, G = geom.shape
    A = aux.shape[1]
    N = P // frames
    FD = G + A
    L1 = W1.shape[1]
    HA = A1.shape[1]
    OUT = W2.shape[1]
    TW = L1 + HA
    H = interleave

    f32 = jnp.float32
    feats = jnp.concatenate(
        [geom, aux, jnp.ones((P, 1), f32)], axis=1)
    geomT = geom.T
    wpre = jnp.zeros((FD + 1, TW), f32)
    wpre = wpre.at[:G, :L1].set(W1[:G] - W1[G:])
    wpre = wpre.at[G:FD, L1:].set(A1[:A])
    wcat = jnp.zeros((FD + 1, TW), f32)
    wcat = wcat.at[:G, :L1].set(W1[G:])
    wcat = wcat.at[G:FD, L1:].set(A1[A:])
    bpre = jnp.concatenate([b1, ab1]).reshape(1, TW)

    body = functools.partial(_pair_body, N, k, G, FD, L1, OUT, H)
    full = lambda i: (0, 0)
    out = pl.pallas_call(
        body,
        grid=(frames // H,),
        in_specs=[
            pl.BlockSpec((H * N, FD + 1), lambda i: (i, 0)),
            pl.BlockSpec((G, H * N), lambda i: (0, i)),
            pl.BlockSpec((FD + 1, TW), full),
            pl.BlockSpec((1, TW), full),
            pl.BlockSpec((FD + 1, TW), full),
            pl.BlockSpec((L1, OUT), full),
            pl.BlockSpec((1, OUT), full),
            pl.BlockSpec((HA, 2 * OUT), full),
            pl.BlockSpec((1, 2 * OUT), full),
            pl.BlockSpec((1, OUT), full),
            pl.BlockSpec((1, OUT), full),
        ],
        out_specs=pl.BlockSpec((H * N, OUT), lambda i: (i, 0)),
        out_shape=jax.ShapeDtypeStruct((P, OUT), f32),
        scratch_shapes=[pltpu.VMEM((N, N), f32) for _ in range(H)],
    )(feats, geomT, wpre, bpre, wcat, W2, b2.reshape(1, OUT), A2,
      ab2.reshape(1, 2 * OUT), ln_g.reshape(1, OUT), ln_b.reshape(1, OUT))
    return out


def kernel(geom, aux, batch, W1, b1, W2, b2, A1, ab1, A2, ab2, ln_g, ln_b):
    del batch
    return _edgeconv(geom, aux, W1, b1, W2, b2, A1, ab1, A2, ab2,
                     ln_g, ln_b, _F, _K, _H)
